# baseline (speedup 1.0000x reference)
import functools

import jax
import jax.numpy as jnp
from jax import lax
from jax.experimental import pallas as pl
from jax.experimental.pallas import tpu as pltpu
from jax.experimental.pallas import tpu_sc as plsc

_ALPHA = 0.95
_LAMBDA = 1.0
_BLOCK_ROWS = 1024
_INT_MIN = -(2 ** 31)
_SC_ROWS = 512
_NW = 32
_L = 16


def _make_sc_kernel(nq, rows, cols):
    rpw = rows // _NW
    mesh = plsc.VectorSubcoreMesh(core_axis_name="c", subcore_axis_name="s")

    @functools.partial(
        pl.kernel, mesh=mesh,
        out_type=jax.ShapeDtypeStruct((_NW, _L), jnp.float32),
        scratch_types=[pltpu.VMEM((cols,), jnp.float32),
                       pltpu.VMEM((cols,), jnp.int32),
                       pltpu.VMEM((_L,), jnp.float32)],
        compiler_params=pltpu.CompilerParams(needs_layout_passes=False),
    )
    def k(x_hbm, out_hbm, row_v, key_v, res_v):
        wid = lax.axis_index("s") * 2 + lax.axis_index("c")
        base = wid * rpw
        nchunk = cols // _L
        zf = jnp.zeros((_L,), jnp.float32)
        of = jnp.ones((_L,), jnp.float32)
        lane0 = lax.iota(jnp.int32, _L) == jnp.zeros((_L,), jnp.int32)

        nq_v = jnp.full((_L,), nq, jnp.int32)
        nq_f = jnp.full((_L,), float(nq), jnp.float32)
        mask7f = jnp.full((_L,), 0x7FFFFFFF, jnp.int32)
        one_v = jnp.ones((_L,), jnp.int32)
        c31 = jnp.full((_L,), 31, jnp.int32)

        def row_body(r, accs):
            a_rowsum, a_sumless, a_corr = accs
            pltpu.sync_copy(x_hbm.at[base + r], row_v)

            def kb(j, c):
                v = row_v[pl.ds(j * _L, _L)]
                b = jax.lax.bitcast_convert_type(v, jnp.int32)
                key_v[pl.ds(j * _L, _L)] = b ^ jnp.bitwise_and(
                    jax.lax.shift_right_arithmetic(b, c31), mask7f)
                return c

            lax.fori_loop(0, nchunk, kb, 0, unroll=8)

            def bstep(it, prefix_v):
                bit_v = jnp.full((_L,), jnp.int32(31) - it, jnp.int32)
                trial_v = prefix_v + jnp.left_shift(one_v, bit_v)

                def cb(j, acc):
                    kv = key_v[pl.ds(j * _L, _L)]
                    return acc + jnp.where(kv < trial_v, one_v,
                                           jnp.zeros((_L,), jnp.int32))

                acc = lax.fori_loop(0, nchunk, cb,
                                    jnp.zeros((_L,), jnp.int32),
                                    unroll=8)
                cnt_v = jnp.full((_L,), jnp.sum(acc), jnp.int32)
                return jnp.where(cnt_v < nq_v, trial_v, prefix_v)

            tkey_v = lax.fori_loop(
                0, 32, bstep, jnp.full((_L,), _INT_MIN, jnp.int32))

            def fb(j, carry):
                s, c, sl = carry
                v = row_v[pl.ds(j * _L, _L)]
                kv = key_v[pl.ds(j * _L, _L)]
                m = kv < tkey_v
                return (s + v, c + jnp.where(m, one_v,
                                             jnp.zeros((_L,), jnp.int32)),
                        sl + jnp.where(m, v, zf))

            s, c, sl = lax.fori_loop(
                0, nchunk, fb, (zf, jnp.zeros((_L,), jnp.int32), zf),
                unroll=8)
            c = jnp.full((_L,), jnp.sum(c), jnp.int32)
            t_bits = tkey_v ^ jnp.bitwise_and(
                jax.lax.shift_right_arithmetic(tkey_v, c31), mask7f)
            t_vec = jax.lax.bitcast_convert_type(t_bits, jnp.float32)
            rem = nq_f - c.astype(jnp.float32)
            corr = t_vec * rem
            return (a_rowsum + s, a_sumless + sl,
                    a_corr + jnp.where(lane0, corr, zf))

        a_rowsum, a_sumless, a_corr = lax.fori_loop(
            0, rpw, row_body, (zf, zf, zf))
        res_v[...] = (-a_rowsum * jnp.float32(1.0 / cols)
                      + (a_sumless + a_corr) * jnp.float32(1.0 / nq))
        pltpu.sync_copy(res_v, out_hbm.at[wid])

    return k


def _keys_of(x):
    bits = jax.lax.bitcast_convert_type(x, jnp.int32)
    return bits ^ jnp.bitwise_and(
        jax.lax.shift_right_arithmetic(bits, 31), jnp.int32(0x7FFFFFFF))


def _cvar_body(nq, x_ref, out_ref, hi_ref, lo_ref):
    i = pl.program_id(0)
    x = x_ref[...]
    rows, cols = x.shape

    keys = _keys_of(x)
    hi_ref[...] = jax.lax.shift_right_arithmetic(keys, 16).astype(jnp.int16)
    lo_ref[...] = (keys ^ jnp.int32(0x8000)).astype(jnp.int16)

    row_sum = jnp.sum(x, axis=1)
    nq16 = jnp.full((1, 1), nq, dtype=jnp.int16)

    def count16(ref, trial):
        m16 = (ref[...] < trial).astype(jnp.int16)
        s = jnp.sum(pltpu.bitcast(m16, jnp.int32), axis=1, keepdims=True)
        return pltpu.bitcast(s, jnp.int16)

    def step16(ref, k_need, it, prefix):
        delta = jnp.left_shift(jnp.int32(1), jnp.int32(15) - it)
        trial = prefix + jnp.broadcast_to(delta, (1, 1)).astype(jnp.int16)
        return jnp.where(count16(ref, trial) < k_need, trial, prefix)

    pa0 = jnp.full((rows, 1), -32768, dtype=jnp.int16)
    h = jax.lax.fori_loop(0, 16, functools.partial(step16, hi_ref, nq16), pa0)

    k2 = nq16 - count16(hi_ref, h)

    lo_ref[...] = jnp.where(hi_ref[...] == h, lo_ref[...],
                            jnp.full((rows, cols), 32767, dtype=jnp.int16))
    l = jax.lax.fori_loop(0, 16, functools.partial(step16, lo_ref, k2), pa0)

    t_key = jnp.left_shift(h.astype(jnp.int32), 16) | jnp.bitwise_and(
        l.astype(jnp.int32) ^ jnp.int32(0x8000), jnp.int32(0xFFFF))

    cnt_less = ((nq16 - k2) + count16(lo_ref, l)).astype(jnp.float32)[:, 0]
    mask = _keys_of(x_ref[...]) < t_key
    sum_less = jnp.sum(jnp.where(mask, x_ref[...], 0.0), axis=1)

    t_bits = t_key ^ jnp.bitwise_and(
        jax.lax.shift_right_arithmetic(t_key, 31), jnp.int32(0x7FFFFFFF))
    t_val = jax.lax.bitcast_convert_type(t_bits, jnp.float32)[:, 0]

    tail_sum = sum_less + t_val * (jnp.float32(nq) - cnt_less)
    loss = -row_sum * jnp.float32(1.0 / cols) + \
        _LAMBDA * tail_sum * jnp.float32(1.0 / nq)
    partial = jnp.sum(loss).reshape(1, 1)

    @pl.when(i == 0)
    def _():
        out_ref[...] = jnp.zeros((1, 1), jnp.float32)

    out_ref[...] += partial


def _tc_call(x, nq, block_rows):
    rows, cols = x.shape
    grid = rows // block_rows
    out = pl.pallas_call(
        functools.partial(_cvar_body, nq),
        grid=(grid,),
        in_specs=[pl.BlockSpec((block_rows, cols), lambda i: (i, 0))],
        out_specs=pl.BlockSpec((1, 1), lambda i: (0, 0)),
        out_shape=jax.ShapeDtypeStruct((1, 1), jnp.float32),
        scratch_shapes=[pltpu.VMEM((block_rows, cols), jnp.int16),
                        pltpu.VMEM((block_rows, cols), jnp.int16)],
    )(x)
    return jnp.reshape(out, ())


def kernel(pred_rets):
    batch, cols = pred_rets.shape
    nq = int(cols * (1 - _ALPHA))
    if nq == 0:
        nq = 1

    sc_rows = _SC_ROWS
    if (batch <= 2 * sc_rows or sc_rows % _NW or cols % _L
            or (batch - sc_rows) % 2):
        sc_rows = 0
    tc_rows = batch - sc_rows
    main_rows = (tc_rows // _BLOCK_ROWS) * _BLOCK_ROWS
    tail_rows = tc_rows - main_rows

    total = jnp.float32(0.0)
    if main_rows:
        total = total + _tc_call(pred_rets[:main_rows], nq, _BLOCK_ROWS)
    if tail_rows:
        total = total + _tc_call(pred_rets[main_rows:tc_rows], nq, tail_rows)
    if sc_rows:
        sc_out = _make_sc_kernel(nq, sc_rows, cols)(pred_rets[tc_rows:])
        total = total + jnp.sum(sc_out)
    return total * jnp.float32(1.0 / batch)


# speedup vs baseline: 1.0159x; 1.0159x over previous
import functools

import jax
import jax.numpy as jnp
from jax import lax
from jax.experimental import pallas as pl
from jax.experimental.pallas import tpu as pltpu
from jax.experimental.pallas import tpu_sc as plsc

_ALPHA = 0.95
_LAMBDA = 1.0
_BLOCK_ROWS = 1024
_INT_MIN = -(2 ** 31)
_SC_ROWS = 896
_NW = 32
_L = 16


def _make_sc_kernel(nq, rows, cols):
    rpw = rows // _NW
    mesh = plsc.VectorSubcoreMesh(core_axis_name="c", subcore_axis_name="s")

    @functools.partial(
        pl.kernel, mesh=mesh,
        out_type=jax.ShapeDtypeStruct((_NW, _L), jnp.float32),
        scratch_types=[pltpu.VMEM((cols,), jnp.float32),
                       pltpu.VMEM((cols,), jnp.int32),
                       pltpu.VMEM((_L,), jnp.float32)],
        compiler_params=pltpu.CompilerParams(needs_layout_passes=False),
    )
    def k(x_hbm, out_hbm, row_v, key_v, res_v):
        wid = lax.axis_index("s") * 2 + lax.axis_index("c")
        base = wid * rpw
        nchunk = cols // _L
        zf = jnp.zeros((_L,), jnp.float32)
        of = jnp.ones((_L,), jnp.float32)
        lane0 = lax.iota(jnp.int32, _L) == jnp.zeros((_L,), jnp.int32)

        nq_v = jnp.full((_L,), nq, jnp.int32)
        nq_f = jnp.full((_L,), float(nq), jnp.float32)
        mask7f = jnp.full((_L,), 0x7FFFFFFF, jnp.int32)
        one_v = jnp.ones((_L,), jnp.int32)
        c31 = jnp.full((_L,), 31, jnp.int32)

        def row_body(r, accs):
            a_rowsum, a_sumless, a_corr = accs
            pltpu.sync_copy(x_hbm.at[base + r], row_v)

            def kb(j, c):
                v = row_v[pl.ds(j * _L, _L)]
                b = jax.lax.bitcast_convert_type(v, jnp.int32)
                key_v[pl.ds(j * _L, _L)] = b ^ jnp.bitwise_and(
                    jax.lax.shift_right_arithmetic(b, c31), mask7f)
                return c

            lax.fori_loop(0, nchunk, kb, 0, unroll=8)

            def bstep(it, prefix_v):
                bit_v = jnp.full((_L,), jnp.int32(31) - it, jnp.int32)
                trial_v = prefix_v + jnp.left_shift(one_v, bit_v)

                def cb(j, acc):
                    kv = key_v[pl.ds(j * _L, _L)]
                    return acc + jnp.where(kv < trial_v, one_v,
                                           jnp.zeros((_L,), jnp.int32))

                acc = lax.fori_loop(0, nchunk, cb,
                                    jnp.zeros((_L,), jnp.int32),
                                    unroll=8)
                cnt_v = jnp.full((_L,), jnp.sum(acc), jnp.int32)
                return jnp.where(cnt_v < nq_v, trial_v, prefix_v)

            tkey_v = lax.fori_loop(
                0, 32, bstep, jnp.full((_L,), _INT_MIN, jnp.int32))

            def fb(j, carry):
                s, c, sl = carry
                v = row_v[pl.ds(j * _L, _L)]
                kv = key_v[pl.ds(j * _L, _L)]
                m = kv < tkey_v
                return (s + v, c + jnp.where(m, one_v,
                                             jnp.zeros((_L,), jnp.int32)),
                        sl + jnp.where(m, v, zf))

            s, c, sl = lax.fori_loop(
                0, nchunk, fb, (zf, jnp.zeros((_L,), jnp.int32), zf),
                unroll=8)
            c = jnp.full((_L,), jnp.sum(c), jnp.int32)
            t_bits = tkey_v ^ jnp.bitwise_and(
                jax.lax.shift_right_arithmetic(tkey_v, c31), mask7f)
            t_vec = jax.lax.bitcast_convert_type(t_bits, jnp.float32)
            rem = nq_f - c.astype(jnp.float32)
            corr = t_vec * rem
            return (a_rowsum + s, a_sumless + sl,
                    a_corr + jnp.where(lane0, corr, zf))

        a_rowsum, a_sumless, a_corr = lax.fori_loop(
            0, rpw, row_body, (zf, zf, zf))
        res_v[...] = (-a_rowsum * jnp.float32(1.0 / cols)
                      + (a_sumless + a_corr) * jnp.float32(1.0 / nq))
        pltpu.sync_copy(res_v, out_hbm.at[wid])

    return k


def _keys_of(x):
    bits = jax.lax.bitcast_convert_type(x, jnp.int32)
    return bits ^ jnp.bitwise_and(
        jax.lax.shift_right_arithmetic(bits, 31), jnp.int32(0x7FFFFFFF))


def _cvar_body(nq, x_ref, out_ref, hi_ref, lo_ref):
    i = pl.program_id(0)
    x = x_ref[...]
    rows, cols = x.shape

    keys = _keys_of(x)
    hi_ref[...] = jax.lax.shift_right_arithmetic(keys, 16).astype(jnp.int16)
    lo_ref[...] = (keys ^ jnp.int32(0x8000)).astype(jnp.int16)

    row_sum = jnp.sum(x, axis=1)
    nq16 = jnp.full((1, 1), nq, dtype=jnp.int16)

    def count16(ref, trial):
        m16 = (ref[...] < trial).astype(jnp.int16)
        s = jnp.sum(pltpu.bitcast(m16, jnp.int32), axis=1, keepdims=True)
        return pltpu.bitcast(s, jnp.int16)

    def step16(ref, k_need, it, prefix):
        delta = jnp.left_shift(jnp.int32(1), jnp.int32(15) - it)
        trial = prefix + jnp.broadcast_to(delta, (1, 1)).astype(jnp.int16)
        return jnp.where(count16(ref, trial) < k_need, trial, prefix)

    pa0 = jnp.full((rows, 1), -32768, dtype=jnp.int16)
    h = jax.lax.fori_loop(0, 16, functools.partial(step16, hi_ref, nq16), pa0)

    k2 = nq16 - count16(hi_ref, h)

    lo_ref[...] = jnp.where(hi_ref[...] == h, lo_ref[...],
                            jnp.full((rows, cols), 32767, dtype=jnp.int16))
    l = jax.lax.fori_loop(0, 16, functools.partial(step16, lo_ref, k2), pa0)

    t_key = jnp.left_shift(h.astype(jnp.int32), 16) | jnp.bitwise_and(
        l.astype(jnp.int32) ^ jnp.int32(0x8000), jnp.int32(0xFFFF))

    cnt_less = ((nq16 - k2) + count16(lo_ref, l)).astype(jnp.float32)[:, 0]
    mask = _keys_of(x_ref[...]) < t_key
    sum_less = jnp.sum(jnp.where(mask, x_ref[...], 0.0), axis=1)

    t_bits = t_key ^ jnp.bitwise_and(
        jax.lax.shift_right_arithmetic(t_key, 31), jnp.int32(0x7FFFFFFF))
    t_val = jax.lax.bitcast_convert_type(t_bits, jnp.float32)[:, 0]

    tail_sum = sum_less + t_val * (jnp.float32(nq) - cnt_less)
    loss = -row_sum * jnp.float32(1.0 / cols) + \
        _LAMBDA * tail_sum * jnp.float32(1.0 / nq)
    partial = jnp.sum(loss).reshape(1, 1)

    @pl.when(i == 0)
    def _():
        out_ref[...] = jnp.zeros((1, 1), jnp.float32)

    out_ref[...] += partial


def _tc_call(x, nq, block_rows):
    rows, cols = x.shape
    grid = rows // block_rows
    out = pl.pallas_call(
        functools.partial(_cvar_body, nq),
        grid=(grid,),
        in_specs=[pl.BlockSpec((block_rows, cols), lambda i: (i, 0))],
        out_specs=pl.BlockSpec((1, 1), lambda i: (0, 0)),
        out_shape=jax.ShapeDtypeStruct((1, 1), jnp.float32),
        scratch_shapes=[pltpu.VMEM((block_rows, cols), jnp.int16),
                        pltpu.VMEM((block_rows, cols), jnp.int16)],
    )(x)
    return jnp.reshape(out, ())


def kernel(pred_rets):
    batch, cols = pred_rets.shape
    nq = int(cols * (1 - _ALPHA))
    if nq == 0:
        nq = 1

    sc_rows = _SC_ROWS
    if (batch <= 2 * sc_rows or sc_rows % _NW or cols % _L
            or (batch - sc_rows) % 2):
        sc_rows = 0
    tc_rows = batch - sc_rows
    main_rows = (tc_rows // _BLOCK_ROWS) * _BLOCK_ROWS
    tail_rows = tc_rows - main_rows

    total = jnp.float32(0.0)
    if sc_rows:
        sc_out = _make_sc_kernel(nq, sc_rows, cols)(pred_rets[tc_rows:])
        total = total + jnp.sum(sc_out)
    if tail_rows:
        total = total + _tc_call(pred_rets[main_rows:tc_rows], nq, tail_rows)
    if main_rows:
        total = total + _tc_call(pred_rets[:main_rows], nq, _BLOCK_ROWS)
    return total * jnp.float32(1.0 / batch)


# speedup vs baseline: 1.0354x; 1.0192x over previous
import functools

import jax
import jax.numpy as jnp
from jax import lax
from jax.experimental import pallas as pl
from jax.experimental.pallas import tpu as pltpu
from jax.experimental.pallas import tpu_sc as plsc

_ALPHA = 0.95
_LAMBDA = 1.0
_BLOCK_ROWS = 1024
_INT_MIN = -(2 ** 31)
_SC_ROWS = 1024
_NW = 32
_L = 16


def _make_sc_kernel(nq, rows, cols):
    rpw = rows // _NW
    mesh = plsc.VectorSubcoreMesh(core_axis_name="c", subcore_axis_name="s")

    @functools.partial(
        pl.kernel, mesh=mesh,
        out_type=jax.ShapeDtypeStruct((_NW, _L), jnp.float32),
        scratch_types=[pltpu.VMEM((cols,), jnp.float32),
                       pltpu.VMEM((cols,), jnp.int32),
                       pltpu.VMEM((_L,), jnp.float32)],
        compiler_params=pltpu.CompilerParams(needs_layout_passes=False),
    )
    def k(x_hbm, out_hbm, row_v, key_v, res_v):
        wid = lax.axis_index("s") * 2 + lax.axis_index("c")
        base = wid * rpw
        nchunk = cols // _L
        zf = jnp.zeros((_L,), jnp.float32)
        of = jnp.ones((_L,), jnp.float32)
        lane0 = lax.iota(jnp.int32, _L) == jnp.zeros((_L,), jnp.int32)

        nq_v = jnp.full((_L,), nq, jnp.int32)
        nq_f = jnp.full((_L,), float(nq), jnp.float32)
        mask7f = jnp.full((_L,), 0x7FFFFFFF, jnp.int32)
        one_v = jnp.ones((_L,), jnp.int32)
        c31 = jnp.full((_L,), 31, jnp.int32)

        def row_body(r, accs):
            a_rowsum, a_sumless, a_corr = accs
            pltpu.sync_copy(x_hbm.at[base + r], row_v)

            def kb(j, c):
                v = row_v[pl.ds(j * _L, _L)]
                b = jax.lax.bitcast_convert_type(v, jnp.int32)
                key_v[pl.ds(j * _L, _L)] = b ^ jnp.bitwise_and(
                    jax.lax.shift_right_arithmetic(b, c31), mask7f)
                return c

            lax.fori_loop(0, nchunk, kb, 0, unroll=8)

            def bstep(it, prefix_v):
                bit_v = jnp.full((_L,), jnp.int32(31) - it, jnp.int32)
                trial_v = prefix_v + jnp.left_shift(one_v, bit_v)

                def cb(j, acc):
                    kv = key_v[pl.ds(j * _L, _L)]
                    return acc + jnp.where(kv < trial_v, one_v,
                                           jnp.zeros((_L,), jnp.int32))

                acc = lax.fori_loop(0, nchunk, cb,
                                    jnp.zeros((_L,), jnp.int32),
                                    unroll=8)
                cnt_v = jnp.full((_L,), jnp.sum(acc), jnp.int32)
                return jnp.where(cnt_v < nq_v, trial_v, prefix_v)

            tkey_v = lax.fori_loop(
                0, 32, bstep, jnp.full((_L,), _INT_MIN, jnp.int32))

            def fb(j, carry):
                s, c, sl = carry
                v = row_v[pl.ds(j * _L, _L)]
                kv = key_v[pl.ds(j * _L, _L)]
                m = kv < tkey_v
                return (s + v, c + jnp.where(m, one_v,
                                             jnp.zeros((_L,), jnp.int32)),
                        sl + jnp.where(m, v, zf))

            s, c, sl = lax.fori_loop(
                0, nchunk, fb, (zf, jnp.zeros((_L,), jnp.int32), zf),
                unroll=8)
            c = jnp.full((_L,), jnp.sum(c), jnp.int32)
            t_bits = tkey_v ^ jnp.bitwise_and(
                jax.lax.shift_right_arithmetic(tkey_v, c31), mask7f)
            t_vec = jax.lax.bitcast_convert_type(t_bits, jnp.float32)
            rem = nq_f - c.astype(jnp.float32)
            corr = t_vec * rem
            return (a_rowsum + s, a_sumless + sl,
                    a_corr + jnp.where(lane0, corr, zf))

        a_rowsum, a_sumless, a_corr = lax.fori_loop(
            0, rpw, row_body, (zf, zf, zf))
        res_v[...] = (-a_rowsum * jnp.float32(1.0 / cols)
                      + (a_sumless + a_corr) * jnp.float32(1.0 / nq))
        pltpu.sync_copy(res_v, out_hbm.at[wid])

    return k


def _keys_of(x):
    bits = jax.lax.bitcast_convert_type(x, jnp.int32)
    return bits ^ jnp.bitwise_and(
        jax.lax.shift_right_arithmetic(bits, 31), jnp.int32(0x7FFFFFFF))


def _cvar_body(nq, x_ref, out_ref, hi_ref, lo_ref):
    i = pl.program_id(0)
    x = x_ref[...]
    rows, cols = x.shape

    keys = _keys_of(x)
    hi_ref[...] = jax.lax.shift_right_arithmetic(keys, 16).astype(jnp.int16)
    lo_ref[...] = (keys ^ jnp.int32(0x8000)).astype(jnp.int16)

    row_sum = jnp.sum(x, axis=1)
    nq16 = jnp.full((1, 1), nq, dtype=jnp.int16)

    def count16(ref, trial):
        m16 = (ref[...] < trial).astype(jnp.int16)
        s = jnp.sum(pltpu.bitcast(m16, jnp.int32), axis=1, keepdims=True)
        return pltpu.bitcast(s, jnp.int16)

    def step16(ref, k_need, it, prefix):
        delta = jnp.left_shift(jnp.int32(1), jnp.int32(15) - it)
        trial = prefix + jnp.broadcast_to(delta, (1, 1)).astype(jnp.int16)
        return jnp.where(count16(ref, trial) < k_need, trial, prefix)

    pa0 = jnp.full((rows, 1), -32768, dtype=jnp.int16)
    h = jax.lax.fori_loop(0, 16, functools.partial(step16, hi_ref, nq16), pa0)

    k2 = nq16 - count16(hi_ref, h)

    lo_ref[...] = jnp.where(hi_ref[...] == h, lo_ref[...],
                            jnp.full((rows, cols), 32767, dtype=jnp.int16))
    l = jax.lax.fori_loop(0, 16, functools.partial(step16, lo_ref, k2), pa0)

    t_key = jnp.left_shift(h.astype(jnp.int32), 16) | jnp.bitwise_and(
        l.astype(jnp.int32) ^ jnp.int32(0x8000), jnp.int32(0xFFFF))

    cnt_less = ((nq16 - k2) + count16(lo_ref, l)).astype(jnp.float32)[:, 0]
    mask = _keys_of(x_ref[...]) < t_key
    sum_less = jnp.sum(jnp.where(mask, x_ref[...], 0.0), axis=1)

    t_bits = t_key ^ jnp.bitwise_and(
        jax.lax.shift_right_arithmetic(t_key, 31), jnp.int32(0x7FFFFFFF))
    t_val = jax.lax.bitcast_convert_type(t_bits, jnp.float32)[:, 0]

    tail_sum = sum_less + t_val * (jnp.float32(nq) - cnt_less)
    loss = -row_sum * jnp.float32(1.0 / cols) + \
        _LAMBDA * tail_sum * jnp.float32(1.0 / nq)
    partial = jnp.sum(loss).reshape(1, 1)

    @pl.when(i == 0)
    def _():
        out_ref[...] = jnp.zeros((1, 1), jnp.float32)

    out_ref[...] += partial


def _tc_call(x, nq, block_rows):
    rows, cols = x.shape
    grid = rows // block_rows
    out = pl.pallas_call(
        functools.partial(_cvar_body, nq),
        grid=(grid,),
        in_specs=[pl.BlockSpec((block_rows, cols), lambda i: (i, 0))],
        out_specs=pl.BlockSpec((1, 1), lambda i: (0, 0)),
        out_shape=jax.ShapeDtypeStruct((1, 1), jnp.float32),
        scratch_shapes=[pltpu.VMEM((block_rows, cols), jnp.int16),
                        pltpu.VMEM((block_rows, cols), jnp.int16)],
    )(x)
    return jnp.reshape(out, ())


def kernel(pred_rets):
    batch, cols = pred_rets.shape
    nq = int(cols * (1 - _ALPHA))
    if nq == 0:
        nq = 1

    sc_rows = _SC_ROWS
    if (batch <= 2 * sc_rows or sc_rows % _NW or cols % _L
            or (batch - sc_rows) % 2):
        sc_rows = 0
    tc_rows = batch - sc_rows
    main_rows = (tc_rows // _BLOCK_ROWS) * _BLOCK_ROWS
    tail_rows = tc_rows - main_rows

    total = jnp.float32(0.0)
    if sc_rows:
        sc_out = _make_sc_kernel(nq, sc_rows, cols)(pred_rets[tc_rows:])
        total = total + jnp.sum(sc_out)
    if tail_rows:
        total = total + _tc_call(pred_rets[main_rows:tc_rows], nq, tail_rows)
    if main_rows:
        total = total + _tc_call(pred_rets[:main_rows], nq, _BLOCK_ROWS)
    return total * jnp.float32(1.0 / batch)
